# baseline (speedup 1.0000x reference)
import jax
import jax.numpy as jnp
from jax import lax
from jax.experimental import pallas as pl

_N = 20000
_C = 91
_IB = 5120
_GRID = (_N + _IB - 1) // _IB


def _loss_body(x_ref, b_ref, lab_ref, t_ref, nll_ref, box_ref, cnt_ref):
    i = pl.program_id(0)

    @pl.when(i == 0)
    def _():
        nll_ref[...] = jnp.zeros((1, 1), jnp.float32)
        box_ref[...] = jnp.zeros((1, 1), jnp.float32)
        cnt_ref[...] = jnp.zeros((1, 1), jnp.float32)

    col = lax.broadcasted_iota(jnp.int32, (1, _IB), 1) + i * _IB
    valid = col < _N

    x = x_ref[...]
    lab = lab_ref[...]

    m = jnp.max(x, axis=0, keepdims=True)
    e = jnp.exp(x - m)
    s = jnp.sum(e, axis=0, keepdims=True)
    lse = jnp.log(s) + m

    rows = lax.broadcasted_iota(jnp.int32, (_C, _IB), 0)
    oh = rows == lab
    sel = jnp.sum(jnp.where(oh, x, 0.0), axis=0, keepdims=True)
    nll_part = jnp.sum(jnp.where(valid, lse - sel, 0.0))
    nll_ref[...] += (nll_part * (1.0 / _N)).reshape(1, 1)

    pos = (lab > 0) & valid
    bpart = jnp.zeros((), jnp.float32)
    for k in range(4):
        bk = b_ref[:, k, :]
        pred_k = jnp.sum(jnp.where(oh, bk, 0.0), axis=0, keepdims=True)
        d = pred_k - t_ref[k:k + 1, :]
        ad = jnp.abs(d)
        el = jnp.where(ad < 1.0, 0.5 * d * d, ad - 0.5)
        bpart += jnp.sum(jnp.where(pos, el, 0.0))
    box_ref[...] += bpart.reshape(1, 1)
    cnt_ref[...] += jnp.sum(jnp.where(pos, 1.0, 0.0)).reshape(1, 1)


def kernel(class_logits, box_regression, labels, regression_targets):
    labels = labels.astype(jnp.int32)
    lt = class_logits.T
    bt = box_regression.transpose(1, 2, 0)
    tt = regression_targets.T
    lab2 = labels.reshape(1, _N)

    nll, bsum, cnt = pl.pallas_call(
        _loss_body,
        grid=(_GRID,),
        in_specs=[
            pl.BlockSpec((_C, _IB), lambda i: (0, i)),
            pl.BlockSpec((_C, 4, _IB), lambda i: (0, 0, i)),
            pl.BlockSpec((1, _IB), lambda i: (0, i)),
            pl.BlockSpec((4, _IB), lambda i: (0, i)),
        ],
        out_specs=[
            pl.BlockSpec((1, 1), lambda i: (0, 0)),
            pl.BlockSpec((1, 1), lambda i: (0, 0)),
            pl.BlockSpec((1, 1), lambda i: (0, 0)),
        ],
        out_shape=[
            jax.ShapeDtypeStruct((1, 1), jnp.float32),
            jax.ShapeDtypeStruct((1, 1), jnp.float32),
            jax.ShapeDtypeStruct((1, 1), jnp.float32),
        ],
    )(lt, bt, lab2, tt)

    ce = nll[0, 0]
    box = bsum[0, 0] / (cnt[0, 0] * 4.0)
    return (ce, box)
